# baseline (speedup 1.0000x reference)
import functools

import jax
import jax.numpy as jnp
from jax import lax
from jax.experimental import pallas as pl
from jax.experimental.pallas import tpu as pltpu
from jax.experimental.pallas import tpu_sc as plsc

NC = 2
NS = 16
NW = NC * NS
CHUNK = 128


def _linear_body(x_ref, w_ref, b_ref, h_ref):
    h_ref[...] = (
        lax.dot_general(
            x_ref[...], w_ref[...], (((1,), (1,)), ((), ())),
            preferred_element_type=jnp.float32,
        )
        + b_ref[...]
    )


def _combine_body(s_ref, c_ref, o_ref):
    cnt = jnp.maximum(c_ref[0] + c_ref[1], 1.0)
    o_ref[...] = jnp.maximum((s_ref[0] + s_ref[1]) / cnt, 0.0)


def _make_scatter(n_pad, nchunk, d):
    rows_pw = n_pad // NS
    assert rows_pw % 8 == 0 and nchunk % 2 == 0
    epw = nchunk * CHUNK
    mesh = plsc.VectorSubcoreMesh(core_axis_name="c", subcore_axis_name="s")

    @functools.partial(
        pl.kernel,
        out_type=[
            jax.ShapeDtypeStruct((NC, n_pad, d), jnp.float32),
            jax.ShapeDtypeStruct((NC * n_pad,), jnp.float32),
        ],
        mesh=mesh,
        scratch_types=[
            pltpu.VMEM((epw,), jnp.int32),
            pltpu.VMEM((CHUNK,), jnp.int32),
            pltpu.VMEM((CHUNK,), jnp.int32),
            pltpu.VMEM((CHUNK, d), jnp.float32),
            pltpu.VMEM((CHUNK, d), jnp.float32),
            pltpu.VMEM((CHUNK,), jnp.float32),
            pltpu.VMEM((n_pad // NS,), jnp.float32),
            pltpu.VMEM_SHARED((n_pad, d), jnp.float32),
            pltpu.VMEM_SHARED((n_pad,), jnp.float32),
            pltpu.SemaphoreType.DMA,
            pltpu.SemaphoreType.DMA,
            pltpu.SemaphoreType.DMA,
            pltpu.SemaphoreType.DMA,
        ],
    )
    def scatter(h_hbm, src_hbm, dst_hbm, zrow_hbm, zcnt_hbm, ones_hbm,
                acc_out, cnt_out,
                src_t, dst_a, dst_b, rows_a, rows_b, ones_v, cnt_v,
                acc_sh, cnt_sh, sem_ra, sem_rb, sem_da, sem_db):
        c = lax.axis_index("c")
        s = lax.axis_index("s")
        wid = s * NC + c

        roff = pl.multiple_of(s * rows_pw, 8)
        base = pl.multiple_of(wid * epw, 8)
        pltpu.sync_copy(zrow_hbm, acc_sh.at[pl.ds(roff, rows_pw)])
        pltpu.sync_copy(zcnt_hbm, cnt_v)
        pltpu.sync_copy(cnt_v, cnt_sh.at[pl.ds(roff, rows_pw)])
        pltpu.sync_copy(ones_hbm, ones_v)
        pltpu.sync_copy(src_hbm.at[pl.ds(base, epw)], src_t)
        plsc.subcore_barrier()

        rows = (rows_a, rows_b)
        dsts = (dst_a, dst_b)
        sems_r = (sem_ra, sem_rb)
        sems_d = (sem_da, sem_db)

        def fire(j, b):
            off = pl.multiple_of(base + j * CHUNK, 8)
            pltpu.async_copy(dst_hbm.at[pl.ds(off, CHUNK)], dsts[b],
                             sems_d[b])
            pltpu.async_copy(h_hbm.at[src_t.at[pl.ds(j * CHUNK, CHUNK)]],
                             rows[b], sems_r[b])

        def wait(b):
            pltpu.make_async_copy(dst_hbm.at[pl.ds(0, CHUNK)], dsts[b],
                                  sems_d[b]).wait()
            pltpu.make_async_copy(h_hbm.at[src_t.at[pl.ds(0, CHUNK)]],
                                  rows[b], sems_r[b]).wait()

        def scat(b):
            pltpu.sync_copy(rows[b], acc_sh.at[dsts[b]], add=True)
            pltpu.sync_copy(ones_v, cnt_sh.at[dsts[b]], add=True)

        fire(0, 0)

        def body(g, carry):
            j = g * 2
            wait(0)
            fire(j + 1, 1)
            scat(0)
            wait(1)
            fire(jnp.minimum(j + 2, nchunk - 1), 0)
            scat(1)
            return carry

        lax.fori_loop(0, nchunk // 2, body, 0)
        wait(0)
        plsc.subcore_barrier()

        pltpu.sync_copy(acc_sh.at[pl.ds(roff, rows_pw)],
                        acc_out.at[c, pl.ds(roff, rows_pw)])
        coff = pl.multiple_of(c * n_pad + roff, 8)
        pltpu.sync_copy(cnt_sh.at[pl.ds(roff, rows_pw)], cnt_v)
        pltpu.sync_copy(cnt_v, cnt_out.at[pl.ds(coff, rows_pw)])

    return scatter


def kernel(x, edge_index, W, b):
    n, d = x.shape
    e = edge_index.shape[1]
    assert d == 128

    n_pad = (n + NS * 8 - 1) // (NS * 8) * (NS * 8)
    blk = 1000
    assert n % blk == 0

    h = pl.pallas_call(
        _linear_body,
        grid=(n // blk,),
        in_specs=[
            pl.BlockSpec((blk, d), lambda i: (i, 0)),
            pl.BlockSpec((d, d), lambda i: (0, 0)),
            pl.BlockSpec((1, d), lambda i: (0, 0)),
        ],
        out_specs=pl.BlockSpec((blk, d), lambda i: (i, 0)),
        out_shape=jax.ShapeDtypeStruct((n, d), jnp.float32),
    )(x, W, b.reshape(1, d))

    step = 2 * CHUNK
    epw = (e + NW * step - 1) // (NW * step) * step
    e_pad = epw * NW
    nchunk = epw // CHUNK
    pad = e_pad - e
    src = jnp.concatenate([edge_index[0], jnp.zeros((pad,), jnp.int32)])
    dst = jnp.concatenate(
        [edge_index[1], jnp.full((pad,), n_pad - 1, jnp.int32)])
    zrow = jnp.zeros((n_pad // NS, d), jnp.float32)
    zcnt = jnp.zeros((n_pad // NS,), jnp.float32)
    ones = jnp.ones((CHUNK,), jnp.float32)
    acc, cnt = _make_scatter(n_pad, nchunk, d)(
        h, src, dst, zrow, zcnt, ones)

    acc_n = acc[:, :n]
    cnt_n = cnt.reshape(NC, n_pad)[:, :n].reshape(NC, n, 1)
    out = pl.pallas_call(
        _combine_body,
        grid=(n // blk,),
        in_specs=[
            pl.BlockSpec((NC, blk, d), lambda i: (0, i, 0)),
            pl.BlockSpec((NC, blk, 1), lambda i: (0, i, 0)),
        ],
        out_specs=pl.BlockSpec((blk, d), lambda i: (i, 0)),
        out_shape=jax.ShapeDtypeStruct((n, d), jnp.float32),
    )(acc_n, cnt_n)
    return out


# speedup vs baseline: 1.4453x; 1.4453x over previous
import functools

import jax
import jax.numpy as jnp
from jax import lax
from jax.experimental import pallas as pl
from jax.experimental.pallas import tpu as pltpu
from jax.experimental.pallas import tpu_sc as plsc

NC = 2
NS = 16
NW = NC * NS
CHUNK = 80


def _linear_body(x_ref, w_ref, b_ref, h_ref):
    h_ref[...] = (
        lax.dot_general(
            x_ref[...], w_ref[...], (((1,), (1,)), ((), ())),
            preferred_element_type=jnp.float32,
        )
        + b_ref[...]
    )


def _combine_body(s_ref, c_ref, o_ref):
    cnt = jnp.maximum(c_ref[0] + c_ref[1], 1.0)
    o_ref[...] = jnp.maximum((s_ref[0] + s_ref[1]) / cnt, 0.0)


def _make_scatter(n_pad, e, d):
    epw = e // NW
    nchunk = epw // CHUNK
    rows_pw = n_pad // NS
    assert rows_pw % 8 == 0
    mesh = plsc.VectorSubcoreMesh(core_axis_name="c", subcore_axis_name="s")

    @functools.partial(
        pl.kernel,
        out_type=[
            jax.ShapeDtypeStruct((NC, n_pad, d), jnp.float32),
            jax.ShapeDtypeStruct((NC * n_pad,), jnp.float32),
        ],
        mesh=mesh,
        scratch_types=[
            pltpu.VMEM((CHUNK,), jnp.int32),
            pltpu.VMEM((CHUNK,), jnp.int32),
            pltpu.VMEM((CHUNK, d), jnp.float32),
            pltpu.VMEM((CHUNK,), jnp.float32),
            pltpu.VMEM((n_pad // NS,), jnp.float32),
            pltpu.VMEM_SHARED((n_pad, d), jnp.float32),
            pltpu.VMEM_SHARED((n_pad,), jnp.float32),
            pltpu.SemaphoreType.DMA,
        ],
    )
    def scatter(h_hbm, src_hbm, dst_hbm, zrow_hbm, zcnt_hbm, ones_hbm,
                acc_out, cnt_out,
                src_v, dst_v, rows_v, ones_v, cnt_v, acc_sh, cnt_sh, sem):
        c = lax.axis_index("c")
        s = lax.axis_index("s")
        wid = s * NC + c

        roff = pl.multiple_of(s * rows_pw, 8)
        pltpu.sync_copy(zrow_hbm, acc_sh.at[pl.ds(roff, rows_pw)])
        pltpu.sync_copy(zcnt_hbm, cnt_v)
        pltpu.sync_copy(cnt_v, cnt_sh.at[pl.ds(roff, rows_pw)])
        pltpu.sync_copy(ones_hbm, ones_v)
        plsc.subcore_barrier()

        base = wid * epw

        def body(i, carry):
            off = pl.multiple_of(base + i * CHUNK, 8)
            pltpu.sync_copy(src_hbm.at[pl.ds(off, CHUNK)], src_v)
            pltpu.sync_copy(dst_hbm.at[pl.ds(off, CHUNK)], dst_v)
            pltpu.async_copy(h_hbm.at[src_v], rows_v, sem).wait()
            pltpu.sync_copy(rows_v, acc_sh.at[dst_v], add=True)
            pltpu.sync_copy(ones_v, cnt_sh.at[dst_v], add=True)
            return carry

        lax.fori_loop(0, nchunk, body, 0)
        plsc.subcore_barrier()

        pltpu.sync_copy(acc_sh.at[pl.ds(roff, rows_pw)],
                        acc_out.at[c, pl.ds(roff, rows_pw)])
        coff = pl.multiple_of(c * n_pad + roff, 8)
        pltpu.sync_copy(cnt_sh.at[pl.ds(roff, rows_pw)], cnt_v)
        pltpu.sync_copy(cnt_v, cnt_out.at[pl.ds(coff, rows_pw)])

    return scatter


def kernel(x, edge_index, W, b):
    n, d = x.shape
    e = edge_index.shape[1]
    assert e % (NW * CHUNK) == 0 and d == 128

    n_pad = (n + NS * 8 - 1) // (NS * 8) * (NS * 8)
    blk = 1000
    assert n % blk == 0

    h = pl.pallas_call(
        _linear_body,
        grid=(n // blk,),
        in_specs=[
            pl.BlockSpec((blk, d), lambda i: (i, 0)),
            pl.BlockSpec((d, d), lambda i: (0, 0)),
            pl.BlockSpec((1, d), lambda i: (0, 0)),
        ],
        out_specs=pl.BlockSpec((blk, d), lambda i: (i, 0)),
        out_shape=jax.ShapeDtypeStruct((n, d), jnp.float32),
    )(x, W, b.reshape(1, d))

    src = edge_index[0]
    dst = edge_index[1]
    zrow = jnp.zeros((n_pad // NS, d), jnp.float32)
    zcnt = jnp.zeros((n_pad // NS,), jnp.float32)
    ones = jnp.ones((CHUNK,), jnp.float32)
    acc, cnt = _make_scatter(n_pad, e, d)(h, src, dst, zrow, zcnt, ones)

    cnt_r = cnt.reshape(NC, n_pad, 1)
    out = pl.pallas_call(
        _combine_body,
        grid=(n // blk,),
        in_specs=[
            pl.BlockSpec((NC, blk, d), lambda i: (0, i, 0)),
            pl.BlockSpec((NC, blk, 1), lambda i: (0, i, 0)),
        ],
        out_specs=pl.BlockSpec((blk, d), lambda i: (i, 0)),
        out_shape=jax.ShapeDtypeStruct((n, d), jnp.float32),
    )(acc, cnt_r)
    return out


# speedup vs baseline: 2.1575x; 1.4928x over previous
import functools

import jax
import jax.numpy as jnp
from jax import lax
from jax.experimental import pallas as pl
from jax.experimental.pallas import tpu as pltpu
from jax.experimental.pallas import tpu_sc as plsc

NC = 2
NS = 16
NW = NC * NS
CHUNK = 80


def _linear_body(x_ref, w_ref, b_ref, h_ref):
    h_ref[...] = (
        lax.dot_general(
            x_ref[...], w_ref[...], (((1,), (1,)), ((), ())),
            preferred_element_type=jnp.float32,
        )
        + b_ref[...]
    )


def _combine_body(s_ref, c_ref, o_ref):
    cnt = jnp.maximum(c_ref[0] + c_ref[1], 1.0)
    o_ref[...] = jnp.maximum((s_ref[0] + s_ref[1]) / cnt, 0.0)


def _make_scatter(n_pad, e, d):
    epw = e // NW
    nchunk = epw // CHUNK
    rows_pw = n_pad // NS
    assert rows_pw % 8 == 0 and nchunk % 2 == 1
    mesh = plsc.VectorSubcoreMesh(core_axis_name="c", subcore_axis_name="s")

    @functools.partial(
        pl.kernel,
        out_type=[
            jax.ShapeDtypeStruct((NC, n_pad, d), jnp.float32),
            jax.ShapeDtypeStruct((NC * n_pad,), jnp.float32),
        ],
        mesh=mesh,
        scratch_types=[
            pltpu.VMEM((CHUNK,), jnp.int32),
            pltpu.VMEM((CHUNK,), jnp.int32),
            pltpu.VMEM((CHUNK,), jnp.int32),
            pltpu.VMEM((CHUNK,), jnp.int32),
            pltpu.VMEM((CHUNK, d), jnp.float32),
            pltpu.VMEM((CHUNK, d), jnp.float32),
            pltpu.VMEM((CHUNK,), jnp.float32),
            pltpu.VMEM((n_pad // NS,), jnp.float32),
            pltpu.VMEM_SHARED((n_pad, d), jnp.float32),
            pltpu.VMEM_SHARED((n_pad,), jnp.float32),
            pltpu.SemaphoreType.DMA,
            pltpu.SemaphoreType.DMA,
        ],
    )
    def scatter(h_hbm, src_hbm, dst_hbm, zrow_hbm, zcnt_hbm, ones_hbm,
                acc_out, cnt_out,
                src_a, src_b, dst_a, dst_b, rows_a, rows_b,
                ones_v, cnt_v, acc_sh, cnt_sh, sem_a, sem_b):
        c = lax.axis_index("c")
        s = lax.axis_index("s")
        wid = s * NC + c

        roff = pl.multiple_of(s * rows_pw, 8)
        pltpu.sync_copy(zrow_hbm, acc_sh.at[pl.ds(roff, rows_pw)])
        pltpu.sync_copy(zcnt_hbm, cnt_v)
        pltpu.sync_copy(cnt_v, cnt_sh.at[pl.ds(roff, rows_pw)])
        pltpu.sync_copy(ones_hbm, ones_v)
        plsc.subcore_barrier()

        base = wid * epw
        srcs = (src_a, src_b)
        dsts = (dst_a, dst_b)
        rows = (rows_a, rows_b)
        sems = (sem_a, sem_b)

        def fire(j, b):
            off = pl.multiple_of(base + j * CHUNK, 8)
            pltpu.sync_copy(src_hbm.at[pl.ds(off, CHUNK)], srcs[b])
            pltpu.sync_copy(dst_hbm.at[pl.ds(off, CHUNK)], dsts[b])
            pltpu.async_copy(h_hbm.at[srcs[b]], rows[b], sems[b])

        def wait(b):
            pltpu.make_async_copy(h_hbm.at[srcs[b]], rows[b], sems[b]).wait()

        def scat(b):
            pltpu.sync_copy(rows[b], acc_sh.at[dsts[b]], add=True)
            pltpu.sync_copy(ones_v, cnt_sh.at[dsts[b]], add=True)

        fire(0, 0)

        def body(g, carry):
            j = g * 2
            fire(j + 1, 1)
            wait(0)
            scat(0)
            fire(jnp.minimum(j + 2, nchunk - 1), 0)
            wait(1)
            scat(1)
            return carry

        lax.fori_loop(0, nchunk // 2, body, 0)
        wait(0)
        scat(0)
        plsc.subcore_barrier()

        pltpu.sync_copy(acc_sh.at[pl.ds(roff, rows_pw)],
                        acc_out.at[c, pl.ds(roff, rows_pw)])
        coff = pl.multiple_of(c * n_pad + roff, 8)
        pltpu.sync_copy(cnt_sh.at[pl.ds(roff, rows_pw)], cnt_v)
        pltpu.sync_copy(cnt_v, cnt_out.at[pl.ds(coff, rows_pw)])

    return scatter


def kernel(x, edge_index, W, b):
    n, d = x.shape
    e = edge_index.shape[1]
    assert e % (NW * CHUNK) == 0 and d == 128

    n_pad = (n + NS * 8 - 1) // (NS * 8) * (NS * 8)
    blk = 1000
    assert n % blk == 0

    h = pl.pallas_call(
        _linear_body,
        grid=(n // blk,),
        in_specs=[
            pl.BlockSpec((blk, d), lambda i: (i, 0)),
            pl.BlockSpec((d, d), lambda i: (0, 0)),
            pl.BlockSpec((1, d), lambda i: (0, 0)),
        ],
        out_specs=pl.BlockSpec((blk, d), lambda i: (i, 0)),
        out_shape=jax.ShapeDtypeStruct((n, d), jnp.float32),
    )(x, W, b.reshape(1, d))

    src = edge_index[0]
    dst = edge_index[1]
    zrow = jnp.zeros((n_pad // NS, d), jnp.float32)
    zcnt = jnp.zeros((n_pad // NS,), jnp.float32)
    ones = jnp.ones((CHUNK,), jnp.float32)
    acc, cnt = _make_scatter(n_pad, e, d)(h, src, dst, zrow, zcnt, ones)

    cnt_r = cnt.reshape(NC, n_pad, 1)
    out = pl.pallas_call(
        _combine_body,
        grid=(n // blk,),
        in_specs=[
            pl.BlockSpec((NC, blk, d), lambda i: (0, i, 0)),
            pl.BlockSpec((NC, blk, 1), lambda i: (0, i, 0)),
        ],
        out_specs=pl.BlockSpec((blk, d), lambda i: (i, 0)),
        out_shape=jax.ShapeDtypeStruct((n, d), jnp.float32),
    )(acc, cnt_r)
    return out
